# Initial kernel scaffold; baseline (speedup 1.0000x reference)
import jax
import jax.numpy as jnp
from jax.experimental import pallas as pl


def kernel(inputs, labels, running_mean, running_var, class_count):
    raise NotImplementedError("write your pallas kernel here")


# speedup vs baseline: 3.4313x; 3.4313x over previous
import functools

import jax
import jax.numpy as jnp
from jax import lax
from jax.experimental import pallas as pl
from jax.experimental.pallas import tpu as pltpu
from jax.experimental.pallas import tpu_sc as plsc

N = 320000
D = 128
C = 10000

NC = 2
NS = 16
HALF = D // NC
ROWS_PER_SUB = N // NS
CHUNK = 80
NCHUNK = ROWS_PER_SUB // CHUNK
CROWS = C // NS
ZROWS = 125


def _sc_body(inputs_hbm, labels_hbm, total_hbm, total2_hbm, counts_hbm,
             idx_v, x_v, sq_v, ones_v, zbuf_v, acc_x, acc_x2, acc_n):
    ci = lax.axis_index("c")
    si = lax.axis_index("s")

    zeros16 = jnp.zeros((16,), jnp.float32)
    one_pat = jnp.where(lax.iota(jnp.int32, 16) == 0, 1.0, 0.0)

    @plsc.parallel_loop(0, ZROWS, unroll=8)
    def _zero(r):
        for c4 in range(HALF // 16):
            zbuf_v[r, pl.ds(16 * c4, 16)] = zeros16

    @plsc.parallel_loop(0, CHUNK, unroll=8)
    def _ones(r):
        ones_v[r, :] = one_pat

    base = si * CROWS
    for z in range(CROWS // ZROWS):
        zb = base + z * ZROWS
        pltpu.sync_copy(zbuf_v, acc_x.at[pl.ds(zb, ZROWS)])
        pltpu.sync_copy(zbuf_v, acc_x2.at[pl.ds(zb, ZROWS)])
        pltpu.sync_copy(zbuf_v.at[:, pl.ds(0, 16)], acc_n.at[pl.ds(zb, ZROWS)])
    plsc.subcore_barrier()

    def chunk_body(j, carry):
        row0 = si * ROWS_PER_SUB + j * CHUNK
        pltpu.sync_copy(labels_hbm.at[si, j], idx_v)
        pltpu.sync_copy(
            inputs_hbm.at[pl.ds(row0, CHUNK), pl.ds(ci * HALF, HALF)], x_v)

        @plsc.parallel_loop(0, CHUNK, unroll=8)
        def _sq(r):
            for c4 in range(HALF // 16):
                v = x_v[r, pl.ds(16 * c4, 16)]
                sq_v[r, pl.ds(16 * c4, 16)] = v * v

        pltpu.sync_copy(x_v, acc_x.at[idx_v], add=True)
        pltpu.sync_copy(sq_v, acc_x2.at[idx_v], add=True)
        pltpu.sync_copy(ones_v, acc_n.at[idx_v], add=True)
        return carry

    lax.fori_loop(0, NCHUNK, chunk_body, 0)
    plsc.subcore_barrier()

    pltpu.sync_copy(acc_x.at[pl.ds(base, CROWS)],
                    total_hbm.at[pl.ds(base, CROWS), pl.ds(ci * HALF, HALF)])
    pltpu.sync_copy(acc_x2.at[pl.ds(base, CROWS)],
                    total2_hbm.at[pl.ds(base, CROWS), pl.ds(ci * HALF, HALF)])

    @pl.when(ci == 0)
    def _():
        pltpu.sync_copy(acc_n.at[pl.ds(base, CROWS)],
                        counts_hbm.at[pl.ds(base, CROWS)])


@jax.jit
def _sc_segment_stats(inputs, labels3):
    mesh = plsc.VectorSubcoreMesh(core_axis_name="c", subcore_axis_name="s")
    f = pl.kernel(
        _sc_body,
        out_type=(
            jax.ShapeDtypeStruct((C, D), jnp.float32),
            jax.ShapeDtypeStruct((C, D), jnp.float32),
            jax.ShapeDtypeStruct((C, 16), jnp.float32),
        ),
        mesh=mesh,
        compiler_params=pltpu.CompilerParams(use_tc_tiling_on_sc=False),
        scratch_types=[
            pltpu.VMEM((CHUNK,), jnp.int32),
            pltpu.VMEM((CHUNK, HALF), jnp.float32),
            pltpu.VMEM((CHUNK, HALF), jnp.float32),
            pltpu.VMEM((CHUNK, 16), jnp.float32),
            pltpu.VMEM((ZROWS, HALF), jnp.float32),
            pltpu.VMEM_SHARED((C, HALF), jnp.float32),
            pltpu.VMEM_SHARED((C, HALF), jnp.float32),
            pltpu.VMEM_SHARED((C, 16), jnp.float32),
        ],
    )
    return f(inputs, labels3)


def _update_body(total_ref, total2_ref, counts_ref, mean_ref, var_ref, cc_ref,
                 new_mean_ref, new_var_ref, new_cc_ref):
    cnt = counts_ref[:, 0:1]
    cc_f = cc_ref[...].astype(jnp.float32)
    inv = 1.0 / (cc_f + cnt)
    keep = cc_f * inv
    new_mean = mean_ref[...] * keep + total_ref[...] * inv
    new_mean_ref[...] = new_mean
    new_var_ref[...] = var_ref[...] * keep + (
        total2_ref[...] - cnt * new_mean * new_mean) * inv
    new_cc_ref[...] = cc_ref[...] + cnt.astype(jnp.int32)


@jax.jit
def _tc_update(total, total2, counts, running_mean, running_var, class_count):
    BC = 1000
    grid = C // BC
    return pl.pallas_call(
        _update_body,
        grid=(grid,),
        in_specs=[
            pl.BlockSpec((BC, D), lambda i: (i, 0)),
            pl.BlockSpec((BC, D), lambda i: (i, 0)),
            pl.BlockSpec((BC, 16), lambda i: (i, 0)),
            pl.BlockSpec((BC, D), lambda i: (i, 0)),
            pl.BlockSpec((BC, D), lambda i: (i, 0)),
            pl.BlockSpec((BC, 1), lambda i: (i, 0)),
        ],
        out_specs=[
            pl.BlockSpec((BC, D), lambda i: (i, 0)),
            pl.BlockSpec((BC, D), lambda i: (i, 0)),
            pl.BlockSpec((BC, 1), lambda i: (i, 0)),
        ],
        out_shape=[
            jax.ShapeDtypeStruct((C, D), jnp.float32),
            jax.ShapeDtypeStruct((C, D), jnp.float32),
            jax.ShapeDtypeStruct((C, 1), jnp.int32),
        ],
    )(total, total2, counts, running_mean, running_var, class_count)


def kernel(inputs, labels, running_mean, running_var, class_count):
    labels3 = labels.reshape(NS, NCHUNK, CHUNK)
    total, total2, counts = _sc_segment_stats(inputs, labels3)
    new_mean, new_var, new_cc = _tc_update(
        total, total2, counts, running_mean, running_var, class_count)
    return new_mean, new_var, new_cc
